# Initial kernel scaffold; baseline (speedup 1.0000x reference)
import jax
import jax.numpy as jnp
from jax.experimental import pallas as pl


def kernel(x, edge_index, W1, b1, W2, b2):
    raise NotImplementedError("write your pallas kernel here")


# speedup vs baseline: 8.1532x; 8.1532x over previous
import dataclasses
import functools

import jax
import jax.numpy as jnp
from jax import lax
from jax.experimental import pallas as pl
from jax.experimental.pallas import tpu as pltpu
from jax.experimental.pallas import tpu_sc as plsc

_N = 10000
_E = 320000
_NC, _NS = 2, 16
_NW = _NC * _NS
_CHUNK = 128
_CPT = 158
_EPAD = _NS * _CPT * _CHUNK
_TRASH = _N
_NACC = _N + 112
_HALF = 5056
_LTRASH = _HALF
_NLOC = _HALF + 64
_LSTRIPE = _NLOC // _NS
_D = 128
_RB = 2000

_mesh = plsc.VectorSubcoreMesh(core_axis_name="c", subcore_axis_name="s")

_cp = pltpu.CompilerParams()
if "needs_layout_passes" in pltpu.CompilerParams.__dataclass_fields__:
    _cp = dataclasses.replace(_cp, needs_layout_passes=False)


@functools.partial(
    pl.kernel,
    out_type=jax.ShapeDtypeStruct((_NW, _NACC), jnp.float32),
    mesh=_mesh,
    compiler_params=_cp,
    scratch_types=[
        pltpu.VMEM((_CPT // 2, _CHUNK), jnp.int32),
        pltpu.VMEM((_NACC,), jnp.float32),
    ],
)
def _deg_sc(dst_hbm, part_hbm, idx_v, acc_v):
    c = lax.axis_index("c")
    s = lax.axis_index("s")
    wid = c * _NS + s

    pltpu.sync_copy(dst_hbm.at[wid], idx_v)

    @pl.loop(0, _NACC, step=16)
    def _(i):
        acc_v[pl.ds(i, 16)] = jnp.zeros((16,), jnp.float32)

    ones = jnp.ones((16,), jnp.float32)

    @pl.loop(0, _CPT // 2)
    def _(j):
        @pl.loop(0, _CHUNK, step=16)
        def _(k):
            idx = idx_v[j, pl.ds(k, 16)]
            plsc.addupdate_scatter(acc_v, [idx], ones)

    pltpu.sync_copy(acc_v, part_hbm.at[wid])


@functools.partial(
    pl.kernel,
    out_type=jax.ShapeDtypeStruct((_NC, _NLOC, _D), jnp.float32),
    mesh=_mesh,
    compiler_params=_cp,
    scratch_types=[
        pltpu.VMEM((_CPT, _CHUNK), jnp.int32),
        pltpu.VMEM((_CPT, _CHUNK), jnp.int32),
        pltpu.VMEM((_CHUNK, _D), jnp.float32),
        pltpu.VMEM_SHARED((_NLOC, _D), jnp.float32),
    ],
)
def _agg_sc(tab_hbm, src_hbm, dst_hbm, zeros_hbm, part_hbm,
            src_v, dst_v, rows_v, acc_sh):
    c = lax.axis_index("c")
    s = lax.axis_index("s")

    pltpu.sync_copy(zeros_hbm.at[pl.ds(s * _LSTRIPE, _LSTRIPE)],
                    acc_sh.at[pl.ds(s * _LSTRIPE, _LSTRIPE)])
    pltpu.sync_copy(src_hbm.at[s], src_v)
    pltpu.sync_copy(dst_hbm.at[s], dst_v)

    base = c * _HALF

    @pl.loop(0, _CPT)
    def _(j):
        @pl.loop(0, _CHUNK, step=16)
        def _(k):
            d = dst_v[j, pl.ds(k, 16)] - base
            ok = (d >= 0) & (d < _HALF)
            dst_v[j, pl.ds(k, 16)] = jnp.where(ok, d, _LTRASH)

    plsc.subcore_barrier()

    @pl.loop(0, _CPT)
    def _(j):
        pltpu.sync_copy(tab_hbm.at[src_v.at[j]], rows_v)
        pltpu.sync_copy(rows_v, acc_sh.at[dst_v.at[j]], add=True)

    plsc.subcore_barrier()
    pltpu.sync_copy(acc_sh.at[pl.ds(s * _LSTRIPE, _LSTRIPE)],
                    part_hbm.at[c].at[pl.ds(s * _LSTRIPE, _LSTRIPE)])


def _dinv_body(dp_ref, o_ref):
    o_ref[...] = lax.rsqrt(jnp.sum(dp_ref[...], axis=0, keepdims=True) + 1.0)


def _tc_dinv(degp):
    return pl.pallas_call(
        _dinv_body,
        grid=(1,),
        in_specs=[pl.BlockSpec((_NW, _NACC), lambda i: (0, 0))],
        out_specs=pl.BlockSpec((1, _NACC), lambda i: (0, 0)),
        out_shape=jax.ShapeDtypeStruct((1, _NACC), jnp.float32),
    )(degp)


def _mm_body(x_ref, w_ref, o_ref):
    o_ref[...] = jnp.dot(x_ref[...], w_ref[...],
                         preferred_element_type=jnp.float32)


def _tc_matmul(x, wt):
    n, din = x.shape
    dout = wt.shape[1]
    return pl.pallas_call(
        _mm_body,
        grid=(n // _RB,),
        in_specs=[pl.BlockSpec((_RB, din), lambda i: (i, 0)),
                  pl.BlockSpec((din, dout), lambda i: (0, 0))],
        out_specs=pl.BlockSpec((_RB, dout), lambda i: (i, 0)),
        out_shape=jax.ShapeDtypeStruct((n, dout), jnp.float32),
    )(x, wt)


def _scale_body(h_ref, dv_ref, o_ref):
    hs = h_ref[...] * dv_ref[...]
    o_ref[...] = jnp.concatenate(
        [hs, jnp.zeros((hs.shape[0], _D - hs.shape[1]), jnp.float32)], axis=1)


def _tc_scale_pad(h, dinv):
    n, d = h.shape
    return pl.pallas_call(
        _scale_body,
        grid=(n // _RB,),
        in_specs=[pl.BlockSpec((_RB, d), lambda i: (i, 0)),
                  pl.BlockSpec((_RB, 1), lambda i: (i, 0))],
        out_specs=pl.BlockSpec((_RB, _D), lambda i: (i, 0)),
        out_shape=jax.ShapeDtypeStruct((n, _D), jnp.float32),
    )(h, dinv)


def _stage2_body(p_ref, hs1_ref, dv_ref, wt_ref, b1_ref, o_ref):
    dinv = dv_ref[...]
    h = (p_ref[...] + hs1_ref[...]) * dinv + b1_ref[...]
    h = jnp.maximum(h, 0.0)
    o_ref[...] = jnp.dot(h, wt_ref[...],
                         preferred_element_type=jnp.float32) * dinv


def _tc_stage2(p, hs1, dinv, wt, b1):
    n, dh = hs1.shape
    do = wt.shape[1]
    return pl.pallas_call(
        _stage2_body,
        grid=(n // _RB,),
        in_specs=[pl.BlockSpec((_RB, dh), lambda i: (i, 0)),
                  pl.BlockSpec((_RB, dh), lambda i: (i, 0)),
                  pl.BlockSpec((_RB, 1), lambda i: (i, 0)),
                  pl.BlockSpec((dh, do), lambda i: (0, 0)),
                  pl.BlockSpec((1, dh), lambda i: (0, 0))],
        out_specs=pl.BlockSpec((_RB, do), lambda i: (i, 0)),
        out_shape=jax.ShapeDtypeStruct((n, do), jnp.float32),
    )(p, hs1, dinv, wt, b1)


def _stage3_body(q_ref, hs2_ref, dv_ref, b2_ref, o_ref):
    o_ref[...] = ((q_ref[...] + hs2_ref[...]) * dv_ref[...] + b2_ref[...])


def _tc_stage3(q, hs2, dinv, b2):
    n, do = hs2.shape
    return pl.pallas_call(
        _stage3_body,
        grid=(n // _RB,),
        in_specs=[pl.BlockSpec((_RB, do), lambda i: (i, 0)),
                  pl.BlockSpec((_RB, do), lambda i: (i, 0)),
                  pl.BlockSpec((_RB, 1), lambda i: (i, 0)),
                  pl.BlockSpec((1, do), lambda i: (0, 0))],
        out_specs=pl.BlockSpec((_RB, do), lambda i: (i, 0)),
        out_shape=jax.ShapeDtypeStruct((n, do), jnp.float32),
    )(q, hs2, dinv, b2)


def _assemble(p):
    return jnp.concatenate([p[0, :_HALF], p[1, :_N - _HALF]], axis=0)


def kernel(x, edge_index, W1, b1, W2, b2):
    src = edge_index[0]
    dst = edge_index[1]
    pad_s = jnp.zeros((_EPAD - _E,), jnp.int32)
    pad_d = jnp.full((_EPAD - _E,), _TRASH, jnp.int32)
    src_p = jnp.concatenate([src, pad_s]).reshape(_NS, _CPT, _CHUNK)
    dst_p = jnp.concatenate([dst, pad_d]).reshape(_NS, _CPT, _CHUNK)
    dst_p32 = dst_p.reshape(_NW, _CPT // 2, _CHUNK)

    degp = _deg_sc(dst_p32)
    dinv = _tc_dinv(degp).reshape(_NACC, 1)[:_N]

    zeros = jnp.zeros((_NLOC, _D), jnp.float32)
    h1 = _tc_matmul(x, W1.T)
    hs1 = _tc_scale_pad(h1, dinv)
    p = _assemble(_agg_sc(hs1, src_p, dst_p, zeros))
    hs2 = _tc_stage2(p[:, :64], hs1[:, :64], dinv,
                     W2.T, b1.reshape(1, -1))
    q = _assemble(_agg_sc(hs2, src_p, dst_p, zeros))
    out = _tc_stage3(q, hs2, dinv, b2.reshape(1, -1))
    return out
